# baseline (speedup 1.0000x reference)
import functools

import jax
import jax.numpy as jnp
from jax import lax
from jax.experimental import pallas as pl
from jax.experimental.pallas import tpu as pltpu
from jax.experimental.pallas import tpu_sc as plsc

_VOXEL = 0.1
_LO = (0.0, -40.0, -3.0)
_GX, _GY, _GZ = 704, 800, 40
_MAX_PTS = 35
_MAX_VOX = 20000
_N = 120000
_PAD = 120832

_TBL = 32768
_TMASK = _TBL - 1
_EMPTY = -1
_CH = 960
_NCH = _N // _CH
_VOX_FLAT = 2883584
_COO_FLAT = 131072
_DUM_VROW = 720000
_DUM_CROW = 20000


def _lin_body(x_ref, y_ref, z_ref, lin_ref):
    x = x_ref[...]
    y = y_ref[...]
    z = z_ref[...]
    cx = jnp.floor((x - _LO[0]) / _VOXEL).astype(jnp.int32)
    cy = jnp.floor((y - _LO[1]) / _VOXEL).astype(jnp.int32)
    cz = jnp.floor((z - _LO[2]) / _VOXEL).astype(jnp.int32)
    valid = ((cx >= 0) & (cx < _GX) & (cy >= 0) & (cy < _GY)
             & (cz >= 0) & (cz < _GZ))
    lin = (cz * _GY + cy) * _GX + cx
    lin_ref[...] = jnp.where(valid, lin, -1)


def _compute_lin(points):
    xyz = jnp.pad(points[:, :3], ((0, _PAD - _N), (0, 0)),
                  constant_values=-1e9)
    x = xyz[:, 0].reshape(944, 128)
    y = xyz[:, 1].reshape(944, 128)
    z = xyz[:, 2].reshape(944, 128)
    lin = pl.pallas_call(
        _lin_body,
        out_shape=jax.ShapeDtypeStruct((944, 128), jnp.int32),
    )(x, y, z)
    return lin.reshape(-1)[:_N]


_mesh = plsc.VectorSubcoreMesh(core_axis_name="c", subcore_axis_name="s",
                               num_cores=1)

_scratch = [
    pltpu.VMEM((8192,), jnp.float32),
    pltpu.VMEM((8192,), jnp.int32),
    pltpu.VMEM((_CH,), jnp.int32),
    pltpu.VMEM((_CH,), jnp.float32),
    pltpu.VMEM((_CH,), jnp.float32),
    pltpu.VMEM((_CH,), jnp.float32),
    pltpu.VMEM((_CH,), jnp.float32),
    pltpu.VMEM((_TBL,), jnp.int32),
    pltpu.VMEM((_TBL,), jnp.int32),
    pltpu.VMEM((20016,), jnp.int32),
    pltpu.VMEM((32,), jnp.int32),
    pltpu.VMEM((16,), jnp.int32),
    pltpu.VMEM((16,), jnp.int32),
    pltpu.VMEM((160,), jnp.float32),
    pltpu.VMEM((160,), jnp.float32),
    pltpu.VMEM((160,), jnp.float32),
    pltpu.VMEM((160,), jnp.float32),
    pltpu.VMEM((160,), jnp.int32),
    pltpu.VMEM((128,), jnp.float32),
    pltpu.VMEM((128,), jnp.float32),
    pltpu.VMEM((128,), jnp.float32),
    pltpu.VMEM((128,), jnp.float32),
    pltpu.VMEM((128,), jnp.int32),
    pltpu.VMEM((128,), jnp.int32),
    pltpu.VMEM((128,), jnp.int32),
    pltpu.VMEM((128,), jnp.int32),
    pltpu.VMEM((160,), jnp.int32),
    pltpu.VMEM((160,), jnp.int32),
    pltpu.VMEM((160,), jnp.int32),
    pltpu.VMEM((160,), jnp.int32),
    pltpu.VMEM((128,), jnp.int32),
    pltpu.VMEM((128,), jnp.int32),
    pltpu.VMEM((128,), jnp.int32),
    pltpu.VMEM((128,), jnp.int32),
    pltpu.VMEM((128,), jnp.int32),
    pltpu.VMEM((128,), jnp.int32),
    pltpu.SemaphoreType.DMA,
    pltpu.SemaphoreType.DMA,
    pltpu.SemaphoreType.DMA,
    pltpu.SemaphoreType.DMA,
]


@functools.partial(
    pl.kernel,
    out_type=[
        jax.ShapeDtypeStruct((_VOX_FLAT,), jnp.float32),
        jax.ShapeDtypeStruct((_COO_FLAT,), jnp.int32),
        jax.ShapeDtypeStruct((_MAX_VOX,), jnp.int32),
    ],
    mesh=_mesh,
    scratch_types=_scratch,
    compiler_params=pltpu.CompilerParams(needs_layout_passes=False),
)
def _sc_voxelize(lin_hbm, px_hbm, py_hbm, pz_hbm, pr_hbm,
                 vox_hbm, coo_hbm, npv_hbm,
                 zf, zi, lin_s, px_s, py_s, pz_s, pr_s,
                 keys, vals, counts, tmp32, tmpa, tmpb,
                 bx, by, bz, br, bidx,
                 dbx, dby, dbz, dbr, di0, di1, di2, di3,
                 ccz, ccy, ccx, cidx, dcz, dcy, dcx, dj0, dj1, dj2,
                 sem_zero, sem_stage, sem_vox, sem_coo):
    sid = lax.axis_index("s")
    lane = lax.iota(jnp.int32, 16)
    fz16 = jnp.zeros((16,), jnp.float32)
    iz16 = jnp.zeros((16,), jnp.int32)

    def _zinit(i, _):
        zf[pl.ds(i * 16, 16)] = fz16
        zi[pl.ds(i * 16, 16)] = iz16
        return 0
    lax.fori_loop(0, 512, _zinit, 0)

    vbase = sid * (22 * 8192)
    for j in range(22):
        pltpu.async_copy(zf, vox_hbm.at[pl.ds(vbase + j * 8192, 8192)],
                         sem_zero)
    pltpu.async_copy(zi, coo_hbm.at[pl.ds(sid * 8192, 8192)], sem_zero)
    for j in range(22):
        pltpu.make_async_copy(zf, vox_hbm.at[pl.ds(vbase + j * 8192, 8192)],
                              sem_zero).wait()
    pltpu.make_async_copy(zi, coo_hbm.at[pl.ds(sid * 8192, 8192)],
                          sem_zero).wait()
    plsc.subcore_barrier()

    @pl.when(sid == 0)
    def _main():
        neg16 = jnp.full((16,), _EMPTY, jnp.int32)

        def _tinit(i, _):
            keys[pl.ds(i * 16, 16)] = neg16
            return 0
        lax.fori_loop(0, _TBL // 16, _tinit, 0)

        def _cinit(i, _):
            counts[pl.ds(i * 16, 16)] = iz16
            return 0
        lax.fori_loop(0, 20016 // 16, _cinit, 0)

        dumv16 = jnp.full((16,), _DUM_VROW, jnp.int32)
        dumc16 = jnp.full((16,), _DUM_CROW, jnp.int32)
        for j in range(10):
            bidx[pl.ds(j * 16, 16)] = dumv16
            cidx[pl.ds(j * 16, 16)] = dumc16

        def _flush_vox(do_wait):
            if do_wait:
                pltpu.make_async_copy(dbx, vox_hbm.at[di0], sem_vox).wait()
                pltpu.make_async_copy(dby, vox_hbm.at[di1], sem_vox).wait()
                pltpu.make_async_copy(dbz, vox_hbm.at[di2], sem_vox).wait()
                pltpu.make_async_copy(dbr, vox_hbm.at[di3], sem_vox).wait()
            for b8 in range(8):
                s = b8 * 16
                dbx[pl.ds(s, 16)] = bx[pl.ds(s, 16)]
                dby[pl.ds(s, 16)] = by[pl.ds(s, 16)]
                dbz[pl.ds(s, 16)] = bz[pl.ds(s, 16)]
                dbr[pl.ds(s, 16)] = br[pl.ds(s, 16)]
                e = bidx[pl.ds(s, 16)] * 4
                di0[pl.ds(s, 16)] = e
                di1[pl.ds(s, 16)] = e + 1
                di2[pl.ds(s, 16)] = e + 2
                di3[pl.ds(s, 16)] = e + 3
            pltpu.async_copy(dbx, vox_hbm.at[di0], sem_vox)
            pltpu.async_copy(dby, vox_hbm.at[di1], sem_vox)
            pltpu.async_copy(dbz, vox_hbm.at[di2], sem_vox)
            pltpu.async_copy(dbr, vox_hbm.at[di3], sem_vox)

        def _do_flush(fill):
            _flush_vox(True)
            bx[pl.ds(0, 16)] = bx[pl.ds(128, 16)]
            by[pl.ds(0, 16)] = by[pl.ds(128, 16)]
            bz[pl.ds(0, 16)] = bz[pl.ds(128, 16)]
            br[pl.ds(0, 16)] = br[pl.ds(128, 16)]
            bidx[pl.ds(0, 16)] = bidx[pl.ds(128, 16)]
            return fill - 128

        _flush_vox(False)

        def _process(v, counter, fill):
            o = v * 16
            lin = lin_s[pl.ds(o, 16)]
            valid = lin >= 0
            key16 = lin * 16 + lane
            res = plsc.sort_key_val(key16, lane)
            sk = res[0] if isinstance(res, (tuple, list)) else res
            tmp32[pl.ds(0, 16)] = jnp.full((16,), -(2 ** 30), jnp.int32)
            tmp32[pl.ds(16, 16)] = jnp.full((16,), 2 ** 30, jnp.int32)
            tmp32[pl.ds(1, 16)] = sk
            prev = tmp32[pl.ds(0, 16)]
            nxt = tmp32[pl.ds(2, 16)]
            slin = sk >> 4
            isnew_run = slin != (prev >> 4)
            islast_s = (slin != (nxt >> 4)).astype(jnp.int32)
            run_start = plsc.cummax(jnp.where(isnew_run, lane, 0))
            lr_sorted = lane - run_start
            orig = sk & 15
            plsc.store_scatter(tmpa, [orig], lr_sorted)
            plsc.store_scatter(tmpb, [orig], islast_s)
            twin_rank = tmpa[...]
            is_last = tmpb[...] == 1
            twin_first = twin_rank == 0

            m = lin * jnp.int32(-1640531527)
            h0 = (lax.shift_right_logical(m, 16) ^ m) & _TMASK
            can_ins = counter < _MAX_VOX

            def _pcond(carry):
                _, unres, _, _ = carry
                return jnp.sum(unres.astype(jnp.int32)) > 0

            def _pbody(carry):
                h, unres, new, drop = carry
                k = plsc.load_gather(keys, [h], mask=unres)
                empty = unres & (k == _EMPTY)
                claim = empty & twin_first & can_ins
                plsc.store_scatter(keys, [h], lin, mask=claim)
                k2 = plsc.load_gather(keys, [h], mask=unres)
                hit2 = unres & (k2 == lin)
                new2 = new | (claim & (k2 == lin))
                drop2 = unres & (k2 == _EMPTY)
                unres2 = unres & ~(hit2 | drop2)
                h2 = jnp.where(unres2, (h + 1) & _TMASK, h)
                return h2, unres2, new2, drop | drop2

            false16 = jnp.zeros((16,), jnp.bool_)
            h_f, _, new, drop = lax.while_loop(
                _pcond, _pbody, (h0, valid, false16, false16))

            newi = new.astype(jnp.int32)
            nnew = jnp.sum(newi)
            slot_new = counter + plsc.cumsum(newi) - newi
            slot_new = jnp.where(slot_new < _MAX_VOX, slot_new, _MAX_VOX)
            plsc.store_scatter(vals, [h_f], slot_new, mask=new)
            live = valid & ~drop
            slot_g = plsc.load_gather(vals, [h_f], mask=live)
            slot = jnp.where(live, slot_g, _MAX_VOX)
            counter2 = jnp.minimum(counter + nnew, _MAX_VOX)

            keepable = valid & (slot < _MAX_VOX)
            slotk = jnp.where(keepable, slot, _MAX_VOX + lane)
            base = plsc.load_gather(counts, [slotk])
            rank = base + twin_rank
            keep = keepable & (rank < _MAX_PTS)
            plsc.store_scatter(counts, [slotk], rank + 1,
                               mask=is_last & keepable)

            keepi = keep.astype(jnp.int32)
            tgt = fill + plsc.cumsum(keepi) - keepi
            vrow = slot * _MAX_PTS + rank
            plsc.store_scatter(bidx, [tgt], vrow, mask=keep)
            plsc.store_scatter(bx, [tgt], px_s[pl.ds(o, 16)], mask=keep)
            plsc.store_scatter(by, [tgt], py_s[pl.ds(o, 16)], mask=keep)
            plsc.store_scatter(bz, [tgt], pz_s[pl.ds(o, 16)], mask=keep)
            plsc.store_scatter(br, [tgt], pr_s[pl.ds(o, 16)], mask=keep)
            return counter2, fill + jnp.sum(keepi)

        def _chunk(c, carry):
            counter, fill = carry
            off = c * _CH
            pltpu.async_copy(lin_hbm.at[pl.ds(off, _CH)], lin_s, sem_stage)
            pltpu.async_copy(px_hbm.at[pl.ds(off, _CH)], px_s, sem_stage)
            pltpu.async_copy(py_hbm.at[pl.ds(off, _CH)], py_s, sem_stage)
            pltpu.async_copy(pz_hbm.at[pl.ds(off, _CH)], pz_s, sem_stage)
            pltpu.async_copy(pr_hbm.at[pl.ds(off, _CH)], pr_s, sem_stage)
            pltpu.make_async_copy(lin_hbm.at[pl.ds(off, _CH)], lin_s,
                                  sem_stage).wait()
            pltpu.make_async_copy(px_hbm.at[pl.ds(off, _CH)], px_s,
                                  sem_stage).wait()
            pltpu.make_async_copy(py_hbm.at[pl.ds(off, _CH)], py_s,
                                  sem_stage).wait()
            pltpu.make_async_copy(pz_hbm.at[pl.ds(off, _CH)], pz_s,
                                  sem_stage).wait()
            pltpu.make_async_copy(pr_hbm.at[pl.ds(off, _CH)], pr_s,
                                  sem_stage).wait()

            def _vbody(v, cr):
                counter, fill = cr
                counter, fill = _process(v, counter, fill)
                fill = lax.cond(fill >= 128, _do_flush, lambda f: f, fill)
                return counter, fill

            return lax.fori_loop(0, _CH // 16, _vbody, (counter, fill))

        counter, fill = lax.fori_loop(
            0, _NCH, _chunk, (jnp.int32(0), jnp.int32(0)))

        _flush_vox(True)
        pltpu.make_async_copy(dbx, vox_hbm.at[di0], sem_vox).wait()
        pltpu.make_async_copy(dby, vox_hbm.at[di1], sem_vox).wait()
        pltpu.make_async_copy(dbz, vox_hbm.at[di2], sem_vox).wait()
        pltpu.make_async_copy(dbr, vox_hbm.at[di3], sem_vox).wait()

        def _flush_coo(do_wait):
            if do_wait:
                pltpu.make_async_copy(dcz, coo_hbm.at[dj0], sem_coo).wait()
                pltpu.make_async_copy(dcy, coo_hbm.at[dj1], sem_coo).wait()
                pltpu.make_async_copy(dcx, coo_hbm.at[dj2], sem_coo).wait()
            for b8 in range(8):
                s = b8 * 16
                dcz[pl.ds(s, 16)] = ccz[pl.ds(s, 16)]
                dcy[pl.ds(s, 16)] = ccy[pl.ds(s, 16)]
                dcx[pl.ds(s, 16)] = ccx[pl.ds(s, 16)]
                e = cidx[pl.ds(s, 16)] * 4
                dj0[pl.ds(s, 16)] = e
                dj1[pl.ds(s, 16)] = e + 1
                dj2[pl.ds(s, 16)] = e + 2
            pltpu.async_copy(dcz, coo_hbm.at[dj0], sem_coo)
            pltpu.async_copy(dcy, coo_hbm.at[dj1], sem_coo)
            pltpu.async_copy(dcx, coo_hbm.at[dj2], sem_coo)

        def _do_flush_coo(fill):
            _flush_coo(True)
            ccz[pl.ds(0, 16)] = ccz[pl.ds(128, 16)]
            ccy[pl.ds(0, 16)] = ccy[pl.ds(128, 16)]
            ccx[pl.ds(0, 16)] = ccx[pl.ds(128, 16)]
            cidx[pl.ds(0, 16)] = cidx[pl.ds(128, 16)]
            return fill - 128

        _flush_coo(False)

        def _cbody(i, cf):
            k = keys[pl.ds(i * 16, 16)]
            vv = vals[pl.ds(i * 16, 16)]
            mm = (k != _EMPTY) & (vv < _MAX_VOX)
            cxv = lax.rem(k, _GX)
            t = lax.div(k, _GX)
            cyv = lax.rem(t, _GY)
            czv = lax.div(t, _GY)
            mi = mm.astype(jnp.int32)
            tgt = cf + plsc.cumsum(mi) - mi
            plsc.store_scatter(ccz, [tgt], czv, mask=mm)
            plsc.store_scatter(ccy, [tgt], cyv, mask=mm)
            plsc.store_scatter(ccx, [tgt], cxv, mask=mm)
            plsc.store_scatter(cidx, [tgt], vv, mask=mm)
            cf = cf + jnp.sum(mi)
            return lax.cond(cf >= 128, _do_flush_coo, lambda f: f, cf)

        lax.fori_loop(0, _TBL // 16, _cbody, jnp.int32(0))
        _flush_coo(True)
        pltpu.make_async_copy(dcz, coo_hbm.at[dj0], sem_coo).wait()
        pltpu.make_async_copy(dcy, coo_hbm.at[dj1], sem_coo).wait()
        pltpu.make_async_copy(dcx, coo_hbm.at[dj2], sem_coo).wait()

        def _nbody(i, _):
            s = i * 16
            counts[pl.ds(s, 16)] = jnp.minimum(counts[pl.ds(s, 16)],
                                               _MAX_PTS)
            return 0
        lax.fori_loop(0, _MAX_VOX // 16, _nbody, 0)
        pltpu.sync_copy(counts.at[pl.ds(0, _MAX_VOX)], npv_hbm)


@jax.jit
def kernel(points):
    lin = _compute_lin(points)
    px = points[:, 0]
    py = points[:, 1]
    pz = points[:, 2]
    pr = points[:, 3]
    vox_f, coo_f, npv = _sc_voxelize(lin, px, py, pz, pr)
    voxels = vox_f[:_MAX_VOX * _MAX_PTS * 4].reshape(_MAX_VOX, _MAX_PTS, 4)
    coors = coo_f.reshape(_COO_FLAT // 4, 4)[:_MAX_VOX, :3]
    return voxels, coors, npv


# speedup vs baseline: 2.4744x; 2.4744x over previous
import functools

import jax
import jax.numpy as jnp
from jax import lax
from jax.experimental import pallas as pl
from jax.experimental.pallas import tpu as pltpu
from jax.experimental.pallas import tpu_sc as plsc

_VOXEL = 0.1
_LO = (0.0, -40.0, -3.0)
_GX, _GY, _GZ = 704, 800, 40
_MAX_PTS = 35
_MAX_VOX = 20000
_N = 120000
_PAD = 120832

_TBL = 32768
_TMASK = _TBL - 1
_EMPTY = -1
_CH = 960
_NCH = _N // _CH
_VOX_FLAT = 2883584
_COO_FLAT = 131072
_DUM_VROW = 720000
_DUM_CROW = 20000


def _lin_body(x_ref, y_ref, z_ref, lin_ref):
    x = x_ref[...]
    y = y_ref[...]
    z = z_ref[...]
    cx = jnp.floor((x - _LO[0]) / _VOXEL).astype(jnp.int32)
    cy = jnp.floor((y - _LO[1]) / _VOXEL).astype(jnp.int32)
    cz = jnp.floor((z - _LO[2]) / _VOXEL).astype(jnp.int32)
    valid = ((cx >= 0) & (cx < _GX) & (cy >= 0) & (cy < _GY)
             & (cz >= 0) & (cz < _GZ))
    lin = (cz * _GY + cy) * _GX + cx
    lin_ref[...] = jnp.where(valid, lin, -1)


def _compute_lin(points):
    xyz = jnp.pad(points[:, :3], ((0, _PAD - _N), (0, 0)),
                  constant_values=-1e9)
    x = xyz[:, 0].reshape(944, 128)
    y = xyz[:, 1].reshape(944, 128)
    z = xyz[:, 2].reshape(944, 128)
    lin = pl.pallas_call(
        _lin_body,
        out_shape=jax.ShapeDtypeStruct((944, 128), jnp.int32),
    )(x, y, z)
    return lin.reshape(-1)[:_N]


_mesh = plsc.VectorSubcoreMesh(core_axis_name="c", subcore_axis_name="s",
                               num_cores=1)

_scratch = [
    pltpu.VMEM((8192,), jnp.float32),
    pltpu.VMEM((8192,), jnp.int32),
    pltpu.VMEM((_CH,), jnp.int32),
    pltpu.VMEM((_CH,), jnp.float32),
    pltpu.VMEM((_CH,), jnp.float32),
    pltpu.VMEM((_CH,), jnp.float32),
    pltpu.VMEM((_CH,), jnp.float32),
    pltpu.VMEM((_TBL,), jnp.int32),
    pltpu.VMEM((_TBL,), jnp.int32),
    pltpu.VMEM((20016,), jnp.int32),
    pltpu.VMEM((32,), jnp.int32),
    pltpu.VMEM((16,), jnp.int32),
    pltpu.VMEM((16,), jnp.int32),
    pltpu.VMEM((160,), jnp.float32),
    pltpu.VMEM((160,), jnp.float32),
    pltpu.VMEM((160,), jnp.float32),
    pltpu.VMEM((160,), jnp.float32),
    pltpu.VMEM((160,), jnp.int32),
    pltpu.VMEM((128,), jnp.float32),
    pltpu.VMEM((128,), jnp.float32),
    pltpu.VMEM((128,), jnp.float32),
    pltpu.VMEM((128,), jnp.float32),
    pltpu.VMEM((128,), jnp.int32),
    pltpu.VMEM((128,), jnp.int32),
    pltpu.VMEM((128,), jnp.int32),
    pltpu.VMEM((128,), jnp.int32),
    pltpu.VMEM((160,), jnp.int32),
    pltpu.VMEM((160,), jnp.int32),
    pltpu.VMEM((160,), jnp.int32),
    pltpu.VMEM((160,), jnp.int32),
    pltpu.VMEM((128,), jnp.int32),
    pltpu.VMEM((128,), jnp.int32),
    pltpu.VMEM((128,), jnp.int32),
    pltpu.VMEM((128,), jnp.int32),
    pltpu.VMEM((128,), jnp.int32),
    pltpu.VMEM((128,), jnp.int32),
    pltpu.SemaphoreType.DMA,
    pltpu.SemaphoreType.DMA,
    pltpu.SemaphoreType.DMA,
    pltpu.SemaphoreType.DMA,
]


@functools.partial(
    pl.kernel,
    out_type=[
        jax.ShapeDtypeStruct((_VOX_FLAT,), jnp.float32),
        jax.ShapeDtypeStruct((_COO_FLAT,), jnp.int32),
        jax.ShapeDtypeStruct((_MAX_VOX,), jnp.int32),
    ],
    mesh=_mesh,
    scratch_types=_scratch,
    compiler_params=pltpu.CompilerParams(needs_layout_passes=False),
)
def _sc_voxelize(lin_hbm, px_hbm, py_hbm, pz_hbm, pr_hbm,
                 vox_hbm, coo_hbm, npv_hbm,
                 zf, zi, lin_s, px_s, py_s, pz_s, pr_s,
                 keys, vals, counts, tmp32, tmpa, tmpb,
                 bx, by, bz, br, bidx,
                 dbx, dby, dbz, dbr, di0, di1, di2, di3,
                 ccz, ccy, ccx, cidx, dcz, dcy, dcx, dj0, dj1, dj2,
                 sem_zero, sem_stage, sem_vox, sem_coo):
    sid = lax.axis_index("s")
    lane = lax.iota(jnp.int32, 16)
    fz16 = jnp.zeros((16,), jnp.float32)
    iz16 = jnp.zeros((16,), jnp.int32)

    def _zinit(i, _):
        zf[pl.ds(i * 16, 16)] = fz16
        zi[pl.ds(i * 16, 16)] = iz16
        return 0
    lax.fori_loop(0, 512, _zinit, 0)

    vbase = sid * (22 * 8192)
    for j in range(22):
        pltpu.async_copy(zf, vox_hbm.at[pl.ds(vbase + j * 8192, 8192)],
                         sem_zero)
    pltpu.async_copy(zi, coo_hbm.at[pl.ds(sid * 8192, 8192)], sem_zero)
    for j in range(22):
        pltpu.make_async_copy(zf, vox_hbm.at[pl.ds(vbase + j * 8192, 8192)],
                              sem_zero).wait()
    pltpu.make_async_copy(zi, coo_hbm.at[pl.ds(sid * 8192, 8192)],
                          sem_zero).wait()
    plsc.subcore_barrier()

    @pl.when(sid == 0)
    def _main():
        neg16 = jnp.full((16,), _EMPTY, jnp.int32)

        def _tinit(i, _):
            keys[pl.ds(i * 16, 16)] = neg16
            return 0
        lax.fori_loop(0, _TBL // 16, _tinit, 0)

        def _cinit(i, _):
            counts[pl.ds(i * 16, 16)] = iz16
            return 0
        lax.fori_loop(0, 20016 // 16, _cinit, 0)

        dumv16 = jnp.full((16,), _DUM_VROW, jnp.int32)
        dumc16 = jnp.full((16,), _DUM_CROW, jnp.int32)
        for j in range(10):
            bidx[pl.ds(j * 16, 16)] = dumv16
            cidx[pl.ds(j * 16, 16)] = dumc16

        def _flush_vox(do_wait):
            if do_wait:
                pltpu.make_async_copy(dbx, vox_hbm.at[di0], sem_vox).wait()
                pltpu.make_async_copy(dby, vox_hbm.at[di1], sem_vox).wait()
                pltpu.make_async_copy(dbz, vox_hbm.at[di2], sem_vox).wait()
                pltpu.make_async_copy(dbr, vox_hbm.at[di3], sem_vox).wait()
            for b8 in range(8):
                s = b8 * 16
                dbx[pl.ds(s, 16)] = bx[pl.ds(s, 16)]
                dby[pl.ds(s, 16)] = by[pl.ds(s, 16)]
                dbz[pl.ds(s, 16)] = bz[pl.ds(s, 16)]
                dbr[pl.ds(s, 16)] = br[pl.ds(s, 16)]
                e = bidx[pl.ds(s, 16)] * 4
                di0[pl.ds(s, 16)] = e
                di1[pl.ds(s, 16)] = e + 1
                di2[pl.ds(s, 16)] = e + 2
                di3[pl.ds(s, 16)] = e + 3
            pltpu.async_copy(dbx, vox_hbm.at[di0], sem_vox)
            pltpu.async_copy(dby, vox_hbm.at[di1], sem_vox)
            pltpu.async_copy(dbz, vox_hbm.at[di2], sem_vox)
            pltpu.async_copy(dbr, vox_hbm.at[di3], sem_vox)

        def _do_flush(fill):
            _flush_vox(True)
            bx[pl.ds(0, 16)] = bx[pl.ds(128, 16)]
            by[pl.ds(0, 16)] = by[pl.ds(128, 16)]
            bz[pl.ds(0, 16)] = bz[pl.ds(128, 16)]
            br[pl.ds(0, 16)] = br[pl.ds(128, 16)]
            bidx[pl.ds(0, 16)] = bidx[pl.ds(128, 16)]
            return fill - 128

        _flush_vox(False)

        def _emit(o, lin, slot, keepable, fill):
            slotk = jnp.where(keepable, slot, _MAX_VOX + lane)
            twin_rank, is_last = plsc.scan_count(slotk, mask=keepable)
            base = plsc.load_gather(counts, [slotk])
            rank = base + twin_rank
            keep = keepable & (rank < _MAX_PTS)
            plsc.store_scatter(counts, [slotk], rank + 1,
                               mask=is_last & keepable)
            keepi = keep.astype(jnp.int32)
            tgt = fill + plsc.cumsum(keepi) - keepi
            vrow = slot * _MAX_PTS + rank
            plsc.store_scatter(bidx, [tgt], vrow, mask=keep)
            plsc.store_scatter(bx, [tgt], px_s[pl.ds(o, 16)], mask=keep)
            plsc.store_scatter(by, [tgt], py_s[pl.ds(o, 16)], mask=keep)
            plsc.store_scatter(bz, [tgt], pz_s[pl.ds(o, 16)], mask=keep)
            plsc.store_scatter(br, [tgt], pr_s[pl.ds(o, 16)], mask=keep)
            return fill + plsc.all_reduce_population_count(keep)[0]

        false16 = jnp.zeros((16,), jnp.bool_)

        def _process(v, counter, fill):
            o = v * 16
            lin = lin_s[pl.ds(o, 16)]
            valid = lin >= 0
            m = lin * jnp.int32(-1640531527)
            h0 = (lax.shift_right_logical(m, 16) ^ m) & _TMASK

            def _heavy(cr):
                counter, fill = cr
                twin_first = plsc.scan_count(lin, mask=valid)[0] == 0

                def _pcond(carry):
                    _, unres, _, _ = carry
                    return plsc.all_reduce_population_count(unres)[0] > 0

                def _pbody(carry):
                    h, unres, new, drop = carry
                    k = plsc.load_gather(keys, [h], mask=unres)
                    empty = unres & (k == _EMPTY)
                    claim = empty & twin_first
                    plsc.store_scatter(keys, [h], lin, mask=claim)
                    k2 = plsc.load_gather(keys, [h], mask=unres)
                    hit2 = unres & (k2 == lin)
                    new2 = new | (claim & (k2 == lin))
                    drop2 = unres & (k2 == _EMPTY)
                    unres2 = unres & ~(hit2 | drop2)
                    h2 = jnp.where(unres2, (h + 1) & _TMASK, h)
                    return h2, unres2, new2, drop | drop2

                h_f, _, new, drop = lax.while_loop(
                    _pcond, _pbody, (h0, valid, false16, false16))

                newi = new.astype(jnp.int32)
                nnew = plsc.all_reduce_population_count(new)[0]
                slot_new = counter + plsc.cumsum(newi) - newi
                slot_new = jnp.where(slot_new < _MAX_VOX, slot_new,
                                     _MAX_VOX)
                plsc.store_scatter(vals, [h_f], slot_new, mask=new)
                live = valid & ~drop
                slot_g = plsc.load_gather(vals, [h_f], mask=live)
                slot = jnp.where(live, slot_g, _MAX_VOX)
                counter2 = jnp.minimum(counter + nnew, _MAX_VOX)
                keepable = valid & (slot < _MAX_VOX)
                return counter2, _emit(o, lin, slot, keepable, fill)

            def _light(cr):
                counter, fill = cr

                def _pcond(carry):
                    _, act, _ = carry
                    return plsc.all_reduce_population_count(act)[0] > 0

                def _pbody(carry):
                    h, act, hit = carry
                    k = plsc.load_gather(keys, [h], mask=act)
                    hitm = act & (k == lin)
                    done = hitm | (act & (k == _EMPTY))
                    act2 = act & ~done
                    h2 = jnp.where(act2, (h + 1) & _TMASK, h)
                    return h2, act2, hit | hitm

                h_f, _, hitm = lax.while_loop(
                    _pcond, _pbody, (h0, valid, false16))

                def _hits(fill):
                    slot_g = plsc.load_gather(vals, [h_f], mask=hitm)
                    slot = jnp.where(hitm, slot_g, _MAX_VOX)
                    keepable = hitm & (slot < _MAX_VOX)
                    return _emit(o, lin, slot, keepable, fill)

                anyhit = plsc.all_reduce_population_count(hitm)[0] > 0
                fill = lax.cond(anyhit, _hits, lambda f: f, fill)
                return counter, fill

            return lax.cond(counter < _MAX_VOX, _heavy, _light,
                            (counter, fill))

        def _chunk(c, carry):
            counter, fill = carry
            off = c * _CH
            pltpu.async_copy(lin_hbm.at[pl.ds(off, _CH)], lin_s, sem_stage)
            pltpu.async_copy(px_hbm.at[pl.ds(off, _CH)], px_s, sem_stage)
            pltpu.async_copy(py_hbm.at[pl.ds(off, _CH)], py_s, sem_stage)
            pltpu.async_copy(pz_hbm.at[pl.ds(off, _CH)], pz_s, sem_stage)
            pltpu.async_copy(pr_hbm.at[pl.ds(off, _CH)], pr_s, sem_stage)
            pltpu.make_async_copy(lin_hbm.at[pl.ds(off, _CH)], lin_s,
                                  sem_stage).wait()
            pltpu.make_async_copy(px_hbm.at[pl.ds(off, _CH)], px_s,
                                  sem_stage).wait()
            pltpu.make_async_copy(py_hbm.at[pl.ds(off, _CH)], py_s,
                                  sem_stage).wait()
            pltpu.make_async_copy(pz_hbm.at[pl.ds(off, _CH)], pz_s,
                                  sem_stage).wait()
            pltpu.make_async_copy(pr_hbm.at[pl.ds(off, _CH)], pr_s,
                                  sem_stage).wait()

            def _vbody(v, cr):
                counter, fill = cr
                counter, fill = _process(v, counter, fill)
                fill = lax.cond(fill >= 128, _do_flush, lambda f: f, fill)
                return counter, fill

            return lax.fori_loop(0, _CH // 16, _vbody, (counter, fill))

        counter, fill = lax.fori_loop(
            0, _NCH, _chunk, (jnp.int32(0), jnp.int32(0)))

        _flush_vox(True)
        pltpu.make_async_copy(dbx, vox_hbm.at[di0], sem_vox).wait()
        pltpu.make_async_copy(dby, vox_hbm.at[di1], sem_vox).wait()
        pltpu.make_async_copy(dbz, vox_hbm.at[di2], sem_vox).wait()
        pltpu.make_async_copy(dbr, vox_hbm.at[di3], sem_vox).wait()

        def _flush_coo(do_wait):
            if do_wait:
                pltpu.make_async_copy(dcz, coo_hbm.at[dj0], sem_coo).wait()
                pltpu.make_async_copy(dcy, coo_hbm.at[dj1], sem_coo).wait()
                pltpu.make_async_copy(dcx, coo_hbm.at[dj2], sem_coo).wait()
            for b8 in range(8):
                s = b8 * 16
                dcz[pl.ds(s, 16)] = ccz[pl.ds(s, 16)]
                dcy[pl.ds(s, 16)] = ccy[pl.ds(s, 16)]
                dcx[pl.ds(s, 16)] = ccx[pl.ds(s, 16)]
                e = cidx[pl.ds(s, 16)] * 4
                dj0[pl.ds(s, 16)] = e
                dj1[pl.ds(s, 16)] = e + 1
                dj2[pl.ds(s, 16)] = e + 2
            pltpu.async_copy(dcz, coo_hbm.at[dj0], sem_coo)
            pltpu.async_copy(dcy, coo_hbm.at[dj1], sem_coo)
            pltpu.async_copy(dcx, coo_hbm.at[dj2], sem_coo)

        def _do_flush_coo(fill):
            _flush_coo(True)
            ccz[pl.ds(0, 16)] = ccz[pl.ds(128, 16)]
            ccy[pl.ds(0, 16)] = ccy[pl.ds(128, 16)]
            ccx[pl.ds(0, 16)] = ccx[pl.ds(128, 16)]
            cidx[pl.ds(0, 16)] = cidx[pl.ds(128, 16)]
            return fill - 128

        _flush_coo(False)

        def _cbody(i, cf):
            k = keys[pl.ds(i * 16, 16)]
            vv = vals[pl.ds(i * 16, 16)]
            mm = (k != _EMPTY) & (vv < _MAX_VOX)
            cxv = lax.rem(k, _GX)
            t = lax.div(k, _GX)
            cyv = lax.rem(t, _GY)
            czv = lax.div(t, _GY)
            mi = mm.astype(jnp.int32)
            tgt = cf + plsc.cumsum(mi) - mi
            plsc.store_scatter(ccz, [tgt], czv, mask=mm)
            plsc.store_scatter(ccy, [tgt], cyv, mask=mm)
            plsc.store_scatter(ccx, [tgt], cxv, mask=mm)
            plsc.store_scatter(cidx, [tgt], vv, mask=mm)
            cf = cf + plsc.all_reduce_population_count(mm)[0]
            return lax.cond(cf >= 128, _do_flush_coo, lambda f: f, cf)

        lax.fori_loop(0, _TBL // 16, _cbody, jnp.int32(0))
        _flush_coo(True)
        pltpu.make_async_copy(dcz, coo_hbm.at[dj0], sem_coo).wait()
        pltpu.make_async_copy(dcy, coo_hbm.at[dj1], sem_coo).wait()
        pltpu.make_async_copy(dcx, coo_hbm.at[dj2], sem_coo).wait()

        def _nbody(i, _):
            s = i * 16
            counts[pl.ds(s, 16)] = jnp.minimum(counts[pl.ds(s, 16)],
                                               _MAX_PTS)
            return 0
        lax.fori_loop(0, _MAX_VOX // 16, _nbody, 0)
        pltpu.sync_copy(counts.at[pl.ds(0, _MAX_VOX)], npv_hbm)


@jax.jit
def kernel(points):
    lin = _compute_lin(points)
    px = points[:, 0]
    py = points[:, 1]
    pz = points[:, 2]
    pr = points[:, 3]
    vox_f, coo_f, npv = _sc_voxelize(lin, px, py, pz, pr)
    voxels = vox_f[:_MAX_VOX * _MAX_PTS * 4].reshape(_MAX_VOX, _MAX_PTS, 4)
    coors = coo_f.reshape(_COO_FLAT // 4, 4)[:_MAX_VOX, :3]
    return voxels, coors, npv
